# baseline (speedup 1.0000x reference)
import functools

import jax
import jax.numpy as jnp
from jax import lax
from jax.experimental import pallas as pl
from jax.experimental.pallas import tpu as pltpu
from jax.experimental.pallas import tpu_sc as plsc

N = 10000
E = 320000
D = 32
NC = 2
NS = 16
NW = NC * NS
EPT = 10240
CB = 128
NCHUNK = EPT // CB
RPT = N // NS
EPAD = NW * EPT - E

def _deg_body(col_hbm, w_hbm, out_hbm, deg_v, col_v, w_v):
    c = lax.axis_index("c")
    s = lax.axis_index("s")
    wid = s * NC + c

    def zero_body(i, _):
        deg_v[pl.ds(i * 16, 16)] = jnp.zeros((16,), jnp.float32)
        return 0

    lax.fori_loop(0, N // 16, zero_body, 0)
    pltpu.sync_copy(col_hbm.at[pl.ds(wid * EPT, EPT)], col_v)
    pltpu.sync_copy(w_hbm.at[pl.ds(wid * EPT, EPT)], w_v)

    def add_body(g, _):
        c16 = col_v[pl.ds(g * 16, 16)]
        w16 = w_v[pl.ds(g * 16, 16)]
        plsc.addupdate_scatter(deg_v, [c16], w16)
        return 0

    lax.fori_loop(0, EPT // 16, add_body, 0)
    pltpu.sync_copy(deg_v, out_hbm.at[wid])


def _mp_body(xs_hbm, row_hbm, col_hbm, w_hbm, zero_hbm, out_hbm,
             ridx_v, cidx_v, w_v, msg0_v, msg1_v, acc_sh, sem0, sem1):
    c = lax.axis_index("c")
    s = lax.axis_index("s")
    wid = s * NC + c

    pltpu.sync_copy(row_hbm.at[wid], ridx_v)
    pltpu.sync_copy(col_hbm.at[wid], cidx_v)
    pltpu.sync_copy(w_hbm.at[pl.ds(wid * EPT, EPT)], w_v)
    pltpu.sync_copy(zero_hbm.at[pl.ds(s * RPT, RPT)],
                    acc_sh.at[pl.ds(s * RPT, RPT)])
    plsc.subcore_barrier()

    def scale(msg_b, j):
        for g in range(CB // 16):
            e16 = jnp.arange(g * 16, g * 16 + 16, dtype=jnp.int32)
            wv = w_v[pl.ds(j * CB + g * 16, 16)]
            for d in range(D):
                d16 = jnp.full((16,), d, jnp.int32)
                vals = plsc.load_gather(msg_b, [e16, d16])
                plsc.store_scatter(msg_b, [e16, d16], vals * wv)

    NJ = NCHUNK // 2
    pltpu.async_copy(xs_hbm.at[ridx_v.at[0]], msg0_v, sem0)

    def chunk_body(jj, _):
        j0 = 2 * jj
        j1 = j0 + 1
        pltpu.async_copy(xs_hbm.at[ridx_v.at[j1]], msg1_v, sem1)
        pltpu.make_async_copy(xs_hbm.at[ridx_v.at[j0]], msg0_v, sem0).wait()
        scale(msg0_v, j0)
        pltpu.sync_copy(msg0_v, acc_sh.at[cidx_v.at[j0]], add=True)

        @pl.when(jj < NJ - 1)
        def _():
            pltpu.async_copy(xs_hbm.at[ridx_v.at[j0 + 2]], msg0_v, sem0)

        pltpu.make_async_copy(xs_hbm.at[ridx_v.at[j1]], msg1_v, sem1).wait()
        scale(msg1_v, j1)
        pltpu.sync_copy(msg1_v, acc_sh.at[cidx_v.at[j1]], add=True)
        return 0

    lax.fori_loop(0, NJ, chunk_body, 0)
    plsc.subcore_barrier()
    pltpu.sync_copy(acc_sh.at[pl.ds(s * RPT, RPT)],
                    out_hbm.at[c, pl.ds(s * RPT, RPT)])


@functools.lru_cache(maxsize=1)
def _sc_kernels():
    mesh = plsc.VectorSubcoreMesh(core_axis_name="c", subcore_axis_name="s",
                                  num_cores=NC, num_subcores=NS)
    params = pltpu.CompilerParams(needs_layout_passes=False,
                                  use_tc_tiling_on_sc=False)
    deg_kernel = pl.kernel(
        _deg_body,
        mesh=mesh,
        compiler_params=params,
        out_type=jax.ShapeDtypeStruct((NW, N), jnp.float32),
        scratch_types=[
            pltpu.VMEM((N,), jnp.float32),
            pltpu.VMEM((EPT,), jnp.int32),
            pltpu.VMEM((EPT,), jnp.float32),
        ],
    )
    mp_kernel = pl.kernel(
        _mp_body,
        mesh=mesh,
        compiler_params=params,
        out_type=jax.ShapeDtypeStruct((NC, N, D), jnp.float32),
        scratch_types=[
            pltpu.VMEM((NCHUNK, CB), jnp.int32),
            pltpu.VMEM((NCHUNK, CB), jnp.int32),
            pltpu.VMEM((EPT,), jnp.float32),
            pltpu.VMEM((CB, D), jnp.float32),
            pltpu.VMEM((CB, D), jnp.float32),
            pltpu.VMEM_SHARED((N, D), jnp.float32),
            pltpu.SemaphoreType.DMA,
            pltpu.SemaphoreType.DMA,
        ],
    )
    return deg_kernel, mp_kernel


def _prep_body(degp_ref, x_ref, w_ref, dinv_ref, xs_ref):
    deg = jnp.sum(degp_ref[...], axis=0) + 1.0
    dinv = lax.rsqrt(deg)[:, None]
    dinv_ref[...] = dinv
    xw = lax.dot_general(x_ref[...], w_ref[...], (((1,), (1,)), ((), ())),
                         preferred_element_type=jnp.float32)
    xs_ref[...] = xw * dinv


def _combine_body(p_ref, xs_ref, dinv_ref, b_ref, g_ref, be_ref, w_ref,
                  xsn_ref):
    dinv = dinv_ref[...]
    t = (p_ref[0] + p_ref[1] + xs_ref[...]) * dinv + b_ref[...]
    m = jnp.mean(t, axis=0, keepdims=True)
    v = jnp.mean((t - m) ** 2, axis=0, keepdims=True)
    h = (t - m) * lax.rsqrt(v + 1e-5) * g_ref[...] + be_ref[...]
    h = jnp.maximum(h, 0.0)
    xw = lax.dot_general(h, w_ref[...], (((1,), (1,)), ((), ())),
                         preferred_element_type=jnp.float32)
    xsn_ref[...] = xw * dinv


def _final_body(p_ref, xs_ref, dinv_ref, b_ref, out_ref):
    t = (p_ref[0] + p_ref[1] + xs_ref[...]) * dinv_ref[...] + b_ref[...]
    out_ref[...] = jnp.maximum(t, 0.0)


def kernel(x, edge_index, edge_weight, W0, b0, W1, b1, W2, b2, W3, b3,
           g0, be0, g1, be1, g2, be2):
    row = jnp.concatenate(
        [edge_index[0], jnp.zeros((EPAD,), jnp.int32)]).reshape(NW, NCHUNK, CB)
    col_flat = jnp.concatenate([edge_index[1], jnp.zeros((EPAD,), jnp.int32)])
    col = col_flat.reshape(NW, NCHUNK, CB)
    w = jnp.concatenate([edge_weight, jnp.zeros((EPAD,), jnp.float32)])
    zero = jnp.zeros((N, D), jnp.float32)

    _deg_kernel, _mp_kernel = _sc_kernels()
    degp = _deg_kernel(col_flat, w)
    dinv, xs = pl.pallas_call(
        _prep_body,
        out_shape=(jax.ShapeDtypeStruct((N, 1), jnp.float32),
                   jax.ShapeDtypeStruct((N, D), jnp.float32)),
    )(degp, x, W0)

    Ws = [W1, W2, W3]
    bs = [b0.reshape(1, D), b1.reshape(1, D), b2.reshape(1, D)]
    gs = [g0.reshape(1, D), g1.reshape(1, D), g2.reshape(1, D)]
    bes = [be0.reshape(1, D), be1.reshape(1, D), be2.reshape(1, D)]
    for l in range(3):
        p = _mp_kernel(xs, row, col, w, zero)
        xs = pl.pallas_call(
            _combine_body,
            out_shape=jax.ShapeDtypeStruct((N, D), jnp.float32),
        )(p, xs, dinv, bs[l], gs[l], bes[l], Ws[l])
    p = _mp_kernel(xs, row, col, w, zero)
    out = pl.pallas_call(
        _final_body,
        out_shape=jax.ShapeDtypeStruct((N, D), jnp.float32),
    )(p, xs, dinv, b3.reshape(1, D))
    return out


# speedup vs baseline: 2.8075x; 2.8075x over previous
import functools

import jax
import jax.numpy as jnp
from jax import lax
from jax.experimental import pallas as pl
from jax.experimental.pallas import tpu as pltpu
from jax.experimental.pallas import tpu_sc as plsc

N = 10000
E = 320000
D = 32
NC = 2
NS = 16
NW = NC * NS
EPT = 10240
CB = 128
NCHUNK = EPT // CB
RPT = N // NS
EPAD = NW * EPT - E

def _deg_body(col_hbm, w_hbm, out_hbm, deg_v, col_v, w_v):
    c = lax.axis_index("c")
    s = lax.axis_index("s")
    wid = s * NC + c

    def zero_body(i, _):
        deg_v[pl.ds(i * 16, 16)] = jnp.zeros((16,), jnp.float32)
        return 0

    lax.fori_loop(0, N // 16, zero_body, 0)
    pltpu.sync_copy(col_hbm.at[pl.ds(wid * EPT, EPT)], col_v)
    pltpu.sync_copy(w_hbm.at[pl.ds(wid * EPT, EPT)], w_v)

    def add_body(g, _):
        c16 = col_v[pl.ds(g * 16, 16)]
        w16 = w_v[pl.ds(g * 16, 16)]
        plsc.addupdate_scatter(deg_v, [c16], w16)
        return 0

    lax.fori_loop(0, EPT // 16, add_body, 0)
    pltpu.sync_copy(deg_v, out_hbm.at[wid])


def _mp_body(xs_hbm, row_hbm, col_hbm, w_hbm, zero_hbm, out_hbm,
             ridx_v, cidx_v, w_v, msg0_v, msg1_v, acc_sh, sem0, sem1):
    c = lax.axis_index("c")
    s = lax.axis_index("s")
    wid = s * NC + c

    pltpu.sync_copy(row_hbm.at[wid], ridx_v)
    pltpu.sync_copy(col_hbm.at[wid], cidx_v)
    pltpu.sync_copy(w_hbm.at[pl.ds(wid * EPT, EPT)], w_v)
    pltpu.sync_copy(zero_hbm.at[pl.ds(s * RPT, RPT)],
                    acc_sh.at[pl.ds(s * RPT, RPT)])
    plsc.subcore_barrier()

    def scale(msg_b, j):
        for g in range(CB // 16):
            wv = w_v[pl.ds(j * CB + g * 16, 16)]
            for e in range(16):
                spl = wv.at[jnp.full((16,), e, jnp.int32)].get(
                    mode="promise_in_bounds")
                er = g * 16 + e
                msg_b[er, pl.ds(0, 16)] = msg_b[er, pl.ds(0, 16)] * spl
                msg_b[er, pl.ds(16, 16)] = msg_b[er, pl.ds(16, 16)] * spl

    NJ = NCHUNK // 2
    pltpu.async_copy(xs_hbm.at[ridx_v.at[0]], msg0_v, sem0)

    def chunk_body(jj, _):
        j0 = 2 * jj
        j1 = j0 + 1
        pltpu.async_copy(xs_hbm.at[ridx_v.at[j1]], msg1_v, sem1)
        pltpu.make_async_copy(xs_hbm.at[ridx_v.at[j0]], msg0_v, sem0).wait()
        scale(msg0_v, j0)
        pltpu.sync_copy(msg0_v, acc_sh.at[cidx_v.at[j0]], add=True)

        @pl.when(jj < NJ - 1)
        def _():
            pltpu.async_copy(xs_hbm.at[ridx_v.at[j0 + 2]], msg0_v, sem0)

        pltpu.make_async_copy(xs_hbm.at[ridx_v.at[j1]], msg1_v, sem1).wait()
        scale(msg1_v, j1)
        pltpu.sync_copy(msg1_v, acc_sh.at[cidx_v.at[j1]], add=True)
        return 0

    lax.fori_loop(0, NJ, chunk_body, 0)
    plsc.subcore_barrier()
    pltpu.sync_copy(acc_sh.at[pl.ds(s * RPT, RPT)],
                    out_hbm.at[c, pl.ds(s * RPT, RPT)])


@functools.lru_cache(maxsize=1)
def _sc_kernels():
    mesh = plsc.VectorSubcoreMesh(core_axis_name="c", subcore_axis_name="s",
                                  num_cores=NC, num_subcores=NS)
    params = pltpu.CompilerParams(needs_layout_passes=False,
                                  use_tc_tiling_on_sc=False)
    deg_kernel = pl.kernel(
        _deg_body,
        mesh=mesh,
        compiler_params=params,
        out_type=jax.ShapeDtypeStruct((NW, N), jnp.float32),
        scratch_types=[
            pltpu.VMEM((N,), jnp.float32),
            pltpu.VMEM((EPT,), jnp.int32),
            pltpu.VMEM((EPT,), jnp.float32),
        ],
    )
    mp_kernel = pl.kernel(
        _mp_body,
        mesh=mesh,
        compiler_params=params,
        out_type=jax.ShapeDtypeStruct((NC, N, D), jnp.float32),
        scratch_types=[
            pltpu.VMEM((NCHUNK, CB), jnp.int32),
            pltpu.VMEM((NCHUNK, CB), jnp.int32),
            pltpu.VMEM((EPT,), jnp.float32),
            pltpu.VMEM((CB, D), jnp.float32),
            pltpu.VMEM((CB, D), jnp.float32),
            pltpu.VMEM_SHARED((N, D), jnp.float32),
            pltpu.SemaphoreType.DMA,
            pltpu.SemaphoreType.DMA,
        ],
    )
    return deg_kernel, mp_kernel


def _prep_body(degp_ref, x_ref, w_ref, dinv_ref, xs_ref):
    deg = jnp.sum(degp_ref[...], axis=0) + 1.0
    dinv = lax.rsqrt(deg)[:, None]
    dinv_ref[...] = dinv
    xw = lax.dot_general(x_ref[...], w_ref[...], (((1,), (1,)), ((), ())),
                         preferred_element_type=jnp.float32)
    xs_ref[...] = xw * dinv


def _combine_body(p_ref, xs_ref, dinv_ref, b_ref, g_ref, be_ref, w_ref,
                  xsn_ref):
    dinv = dinv_ref[...]
    t = (p_ref[0] + p_ref[1] + xs_ref[...]) * dinv + b_ref[...]
    m = jnp.mean(t, axis=0, keepdims=True)
    v = jnp.mean((t - m) ** 2, axis=0, keepdims=True)
    h = (t - m) * lax.rsqrt(v + 1e-5) * g_ref[...] + be_ref[...]
    h = jnp.maximum(h, 0.0)
    xw = lax.dot_general(h, w_ref[...], (((1,), (1,)), ((), ())),
                         preferred_element_type=jnp.float32)
    xsn_ref[...] = xw * dinv


def _final_body(p_ref, xs_ref, dinv_ref, b_ref, out_ref):
    t = (p_ref[0] + p_ref[1] + xs_ref[...]) * dinv_ref[...] + b_ref[...]
    out_ref[...] = jnp.maximum(t, 0.0)


def kernel(x, edge_index, edge_weight, W0, b0, W1, b1, W2, b2, W3, b3,
           g0, be0, g1, be1, g2, be2):
    row = jnp.concatenate(
        [edge_index[0], jnp.zeros((EPAD,), jnp.int32)]).reshape(NW, NCHUNK, CB)
    col_flat = jnp.concatenate([edge_index[1], jnp.zeros((EPAD,), jnp.int32)])
    col = col_flat.reshape(NW, NCHUNK, CB)
    w = jnp.concatenate([edge_weight, jnp.zeros((EPAD,), jnp.float32)])
    zero = jnp.zeros((N, D), jnp.float32)

    _deg_kernel, _mp_kernel = _sc_kernels()
    degp = _deg_kernel(col_flat, w)
    dinv, xs = pl.pallas_call(
        _prep_body,
        out_shape=(jax.ShapeDtypeStruct((N, 1), jnp.float32),
                   jax.ShapeDtypeStruct((N, D), jnp.float32)),
    )(degp, x, W0)

    Ws = [W1, W2, W3]
    bs = [b0.reshape(1, D), b1.reshape(1, D), b2.reshape(1, D)]
    gs = [g0.reshape(1, D), g1.reshape(1, D), g2.reshape(1, D)]
    bes = [be0.reshape(1, D), be1.reshape(1, D), be2.reshape(1, D)]
    for l in range(3):
        p = _mp_kernel(xs, row, col, w, zero)
        xs = pl.pallas_call(
            _combine_body,
            out_shape=jax.ShapeDtypeStruct((N, D), jnp.float32),
        )(p, xs, dinv, bs[l], gs[l], bes[l], Ws[l])
    p = _mp_kernel(xs, row, col, w, zero)
    out = pl.pallas_call(
        _final_body,
        out_shape=jax.ShapeDtypeStruct((N, D), jnp.float32),
    )(p, xs, dinv, b3.reshape(1, D))
    return out
